# baseline (speedup 1.0000x reference)
import functools

import jax
import jax.numpy as jnp
from jax import lax
from jax.experimental import pallas as pl
from jax.experimental.pallas import tpu as pltpu
from jax.experimental.pallas import tpu_sc as plsc

_NC = 2
_NS = 16
_NW = _NC * _NS
_B = 16384
_D = 64
_BPW = _B // _NW
_CH = 128
_NCH = _BPW // _CH


def _build():
    mesh = plsc.VectorSubcoreMesh(core_axis_name="c", subcore_axis_name="s")

    @functools.partial(
        pl.kernel,
        mesh=mesh,
        out_type=jax.ShapeDtypeStruct((_B, _D), jnp.float32),
        compiler_params=pltpu.CompilerParams(use_tc_tiling_on_sc=False),
        scratch_types=[
            pltpu.VMEM((_NCH, _CH), jnp.int32),
            pltpu.VMEM((_BPW, _D), jnp.float32),
            pltpu.SemaphoreType.DMA,
        ],
    )
    def gather_kernel(idx_hbm, table_hbm, out_hbm, idx_v, rows_v, sem):
        wid = lax.axis_index("s") * _NC + lax.axis_index("c")
        pltpu.sync_copy(idx_hbm.at[wid], idx_v)
        copies = [
            pltpu.async_copy(
                table_hbm.at[idx_v.at[j]],
                rows_v.at[pl.ds(j * _CH, _CH)],
                sem,
            )
            for j in range(_NCH)
        ]
        for c in copies:
            c.wait()
        pltpu.sync_copy(rows_v, out_hbm.at[pl.ds(wid * _BPW, _BPW)])

    return gather_kernel


_gather = _build()


def kernel(input_, dim, index):
    idx = (index + jnp.asarray(dim, index.dtype)).reshape(_NW, _NCH, _CH)
    return _gather(idx, input_)


# speedup vs baseline: 1.4216x; 1.4216x over previous
import functools

import jax
import jax.numpy as jnp
from jax import lax
from jax.experimental import pallas as pl
from jax.experimental.pallas import tpu as pltpu
from jax.experimental.pallas import tpu_sc as plsc

_NC = 2
_NS = 16
_NW = _NC * _NS
_B = 16384
_D = 64
_BPW = _B // _NW
_G = 16
_NG = _BPW // _G


def _build():
    mesh = plsc.VectorSubcoreMesh(core_axis_name="c", subcore_axis_name="s")

    @functools.partial(
        pl.kernel,
        mesh=mesh,
        out_type=jax.ShapeDtypeStruct((_B, _D), jnp.float32),
        compiler_params=pltpu.CompilerParams(
            use_tc_tiling_on_sc=True, needs_layout_passes=False
        ),
        scratch_types=[
            pltpu.VMEM((4, 128), jnp.int32),
            pltpu.VMEM((_G, 8, _D), jnp.float32),
            pltpu.VMEM((_BPW, _D), jnp.float32),
            pltpu.SemaphoreType.DMA,
        ],
    )
    def gather_kernel(idx_hbm, table_hbm, out_hbm, idx_v, slab_v, rows_v, sem):
        wid = lax.axis_index("s") * _NC + lax.axis_index("c")
        pltpu.sync_copy(idx_hbm.at[wid], idx_v)
        lanes16 = lax.broadcasted_iota(jnp.int32, (16,), 0)

        def group_body(g, _):
            vec = idx_v[g // 8, pl.ds((g % 8) * 16, 16)]
            copies = []
            for j in range(_G):
                base = pl.multiple_of((vec[j] // 8) * 8, 8)
                copies.append(
                    pltpu.async_copy(
                        table_hbm.at[pl.ds(base, 8), :], slab_v.at[j], sem
                    )
                )
            for c in copies:
                c.wait()
            sub = vec & 7
            rowvec = g * _G + lanes16
            for col in range(_D):
                vals = plsc.load_gather(
                    slab_v, [lanes16, sub, jnp.full((16,), col, jnp.int32)]
                )
                plsc.store_scatter(
                    rows_v, [rowvec, jnp.full((16,), col, jnp.int32)], vals
                )
            return ()

        lax.fori_loop(0, _NG, group_body, ())
        pltpu.sync_copy(rows_v, out_hbm.at[pl.ds(wid * _BPW, _BPW)])

    return gather_kernel


_gather = _build()


def kernel(input_, dim, index):
    idx = (index + jnp.asarray(dim, index.dtype)).reshape(_NW, 4, 128)
    return _gather(idx, input_)


# speedup vs baseline: 1.9537x; 1.3743x over previous
import functools

import jax
import jax.numpy as jnp
from jax import lax
from jax.experimental import pallas as pl
from jax.experimental.pallas import tpu as pltpu
from jax.experimental.pallas import tpu_sc as plsc

_NC = 2
_NS = 16
_NW = _NC * _NS
_B = 16384
_D = 64
_BPW = _B // _NW
_G = 16
_NG = _BPW // _G


def _build():
    mesh = plsc.VectorSubcoreMesh(core_axis_name="c", subcore_axis_name="s")

    @functools.partial(
        pl.kernel,
        mesh=mesh,
        out_type=jax.ShapeDtypeStruct((_B, _D), jnp.float32),
        compiler_params=pltpu.CompilerParams(
            use_tc_tiling_on_sc=True, needs_layout_passes=False
        ),
        scratch_types=[
            pltpu.VMEM((4, 128), jnp.int32),
            pltpu.VMEM((_G, 8, _D), jnp.float32),
            pltpu.VMEM((_BPW, _D), jnp.float32),
            pltpu.SemaphoreType.DMA,
        ],
    )
    def gather_kernel(idx_hbm, table_hbm, out_hbm, idx_v, slab_v, rows_v, sem):
        wid = lax.axis_index("s") * _NC + lax.axis_index("c")
        pltpu.sync_copy(idx_hbm.at[wid], idx_v)
        lanes16 = lax.broadcasted_iota(jnp.int32, (16,), 0)

        def group_body(g, _):
            vec = idx_v[g // 8, pl.ds((g % 8) * 16, 16)]
            copies = []
            for j in range(_G):
                base = pl.multiple_of((vec[j] // 8) * 8, 8)
                copies.append(
                    pltpu.async_copy(
                        table_hbm.at[pl.ds(base, 8), :], slab_v.at[j], sem
                    )
                )
            for c in copies:
                c.wait()
            sub = vec & 7
            rowvec = g * _G + lanes16
            for col in range(_D):
                vals = plsc.load_gather(
                    slab_v, [lanes16, sub, jnp.full((16,), col, jnp.int32)]
                )
                plsc.store_scatter(
                    rows_v, [rowvec, jnp.full((16,), col, jnp.int32)], vals
                )
            return ()

        lax.fori_loop(0, _NG, group_body, ())
        pltpu.sync_copy(rows_v, out_hbm.at[pl.ds(wid * _BPW, _BPW)])

    return gather_kernel


_gather = _build()


def kernel(input_, dim, index):
    idx = (index + jnp.asarray(dim, index.dtype)).reshape(_NW, 4, 128)
    table = jnp.swapaxes(lax.optimization_barrier(jnp.swapaxes(input_, 0, 1)), 0, 1)
    return _gather(idx, table)


# speedup vs baseline: 2.1882x; 1.1200x over previous
import functools

import jax
import jax.numpy as jnp
from jax import lax
from jax.experimental import pallas as pl
from jax.experimental.pallas import tpu as pltpu
from jax.experimental.pallas import tpu_sc as plsc

_NC = 2
_NS = 16
_NW = _NC * _NS
_B = 16384
_D = 64
_BPW = _B // _NW
_G = 16
_NG = _BPW // _G


def _build():
    mesh = plsc.VectorSubcoreMesh(core_axis_name="c", subcore_axis_name="s")

    @functools.partial(
        pl.kernel,
        mesh=mesh,
        out_type=jax.ShapeDtypeStruct((_B, _D), jnp.float32),
        compiler_params=pltpu.CompilerParams(
            use_tc_tiling_on_sc=True, needs_layout_passes=False
        ),
        scratch_types=[
            pltpu.VMEM((4, 128), jnp.int32),
            pltpu.VMEM((2, _G, 8, _D), jnp.float32),
            pltpu.VMEM((_BPW, _D), jnp.float32),
            pltpu.SemaphoreType.DMA,
            pltpu.SemaphoreType.DMA,
        ],
    )
    def gather_kernel(
        idx_hbm, table_hbm, out_hbm, idx_v, slab_v, rows_v, sem_a, sem_b
    ):
        wid = lax.axis_index("s") * _NC + lax.axis_index("c")
        pltpu.sync_copy(idx_hbm.at[wid], idx_v)
        lanes16 = lax.broadcasted_iota(jnp.int32, (16,), 0)

        def load_vec(g):
            return idx_v[g // 8, pl.ds((g % 8) * 16, 16)]

        def fire(g, buf, sem):
            vec = load_vec(g)
            for j in range(_G):
                base = pl.multiple_of((vec[j] // 8) * 8, 8)
                pltpu.async_copy(
                    table_hbm.at[pl.ds(base, 8), :], slab_v.at[buf, j], sem
                )

        def drain(buf, sem):
            for j in range(_G):
                pltpu.make_async_copy(
                    table_hbm.at[pl.ds(0, 8), :], slab_v.at[buf, j], sem
                ).wait()

        def extract(g, buf):
            vec = load_vec(g)
            sub = vec & 7
            rowvec = g * _G + lanes16
            for col in range(_D):
                vals = plsc.load_gather(
                    slab_v.at[buf],
                    [lanes16, sub, jnp.full((16,), col, jnp.int32)],
                )
                plsc.store_scatter(
                    rows_v, [rowvec, jnp.full((16,), col, jnp.int32)], vals
                )

        fire(0, 0, sem_a)

        def pair_body(p, _):
            g = p * 2
            fire(g + 1, 1, sem_b)
            drain(0, sem_a)
            extract(g, 0)

            @pl.when(p < _NG // 2 - 1)
            def _():
                fire(g + 2, 0, sem_a)

            drain(1, sem_b)
            extract(g + 1, 1)
            return ()

        lax.fori_loop(0, _NG // 2, pair_body, ())
        pltpu.sync_copy(rows_v, out_hbm.at[pl.ds(wid * _BPW, _BPW)])

    return gather_kernel


_gather = _build()


def kernel(input_, dim, index):
    idx = (index + jnp.asarray(dim, index.dtype)).reshape(_NW, 4, 128)
    table = jnp.swapaxes(lax.optimization_barrier(jnp.swapaxes(input_, 0, 1)), 0, 1)
    return _gather(idx, table)
